# baseline (speedup 1.0000x reference)
import functools

import jax
import jax.numpy as jnp
import numpy as np
from jax import lax
from jax.experimental import pallas as pl
from jax.experimental.pallas import tpu as pltpu
from jax.experimental.pallas import tpu_sc as plsc

N = 10000
E = 320000
D = 128

NC = 2
NS = 16
NW = NC * NS
EPT = E // NW
DEG_CHUNK = 2000
AGG_CHUNK = 80
ROW_CHUNK = 80
N_ROW_CHUNKS = N // ROW_CHUNK

_BN_SCALE = float(1.0 / np.sqrt(1.0 + 1e-5))

_MESH = plsc.VectorSubcoreMesh(core_axis_name="c", subcore_axis_name="s")


@functools.partial(
    pl.kernel,
    out_type=jax.ShapeDtypeStruct((NW * N,), jnp.float32),
    mesh=_MESH,
    scratch_types=[
        pltpu.VMEM((DEG_CHUNK,), jnp.int32),
        pltpu.VMEM((N,), jnp.float32),
    ],
    compiler_params=pltpu.CompilerParams(needs_layout_passes=False),
)
def _sc_degree(dst_hbm, out_hbm, dst_v, hist_v):
    g = lax.axis_index("c") * NS + lax.axis_index("s")

    def zero_body(i, carry):
        hist_v[pl.ds(i * 16, 16)] = jnp.zeros((16,), jnp.float32)
        return carry

    lax.fori_loop(0, N // 16, zero_body, 0)

    ones = jnp.ones((16,), jnp.float32)

    def chunk_body(ci, carry):
        pltpu.sync_copy(dst_hbm.at[pl.ds(g * EPT + ci * DEG_CHUNK, DEG_CHUNK)],
                        dst_v)

        def inner(j, c2):
            idx = dst_v[pl.ds(j * 16, 16)]
            plsc.addupdate_scatter(hist_v, [idx], ones)
            return c2

        lax.fori_loop(0, DEG_CHUNK // 16, inner, 0)
        return carry

    lax.fori_loop(0, EPT // DEG_CHUNK, chunk_body, 0)
    pltpu.sync_copy(hist_v, out_hbm.at[pl.ds(g * N, N)])


NBUF = 3
ECHUNK = 96
NCHUNK = 105
EPAD = NW * NCHUNK * ECHUNK
_CHUNK_BYTES = ECHUNK * D * 4


@functools.partial(
    pl.kernel,
    out_type=jax.ShapeDtypeStruct((NC * N, D), jnp.float32),
    mesh=_MESH,
    scratch_types=[
        pltpu.VMEM_SHARED((N + 8, D), jnp.float32),
        pltpu.VMEM((NCHUNK * ECHUNK,), jnp.int32),
        [pltpu.VMEM((ECHUNK,), jnp.int32) for _ in range(NBUF)],
        [pltpu.VMEM((ECHUNK,), jnp.int32) for _ in range(NBUF)],
        pltpu.VMEM((NBUF, ECHUNK, D), jnp.float32),
        [pltpu.SemaphoreType.DMA] * NBUF,
        [pltpu.SemaphoreType.DMA] * NBUF,
    ],
)
def _sc_aggregate(u_hbm, pk_hbm, zeros_hbm, out_hbm,
                  s_sh, pk_st, src_v, dst_v, rows_v, gsems, ssems):
    c = lax.axis_index("c")
    s = lax.axis_index("s")
    g = c * NS + s

    pltpu.sync_copy(pk_hbm.at[g], pk_st)

    def zero_body(j, carry):
        cid = s + j * NS

        @pl.when(cid < N_ROW_CHUNKS)
        def _():
            pltpu.sync_copy(zeros_hbm, s_sh.at[pl.ds(cid * ROW_CHUNK,
                                                     ROW_CHUNK)])
        return carry

    lax.fori_loop(0, (N_ROW_CHUNKS + NS - 1) // NS, zero_body, 0)
    plsc.subcore_barrier()

    def unpack(ci, b):
        for k in range(ECHUNK // 16):
            v = pk_st[pl.ds(ci * ECHUNK + k * 16, 16)]
            src_v[b][pl.ds(k * 16, 16)] = lax.bitwise_and(v, 0xFFFF)
            dst_v[b][pl.ds(k * 16, 16)] = lax.shift_right_logical(v, 16)

    def outer_body(o, carry):
        for b in range(NBUF):
            ci = o * NBUF + b

            @pl.when(o > 0)
            def _():
                pltpu.make_async_copy(rows_v.at[b], s_sh.at[dst_v[b]],
                                      ssems[b]).wait()
            unpack(ci, b)
            pltpu.async_copy(u_hbm.at[src_v[b]], rows_v.at[b], gsems[b])
        for b in range(NBUF):
            pltpu.make_async_copy(u_hbm.at[src_v[b]], rows_v.at[b],
                                  gsems[b]).wait()
            pltpu.async_copy(rows_v.at[b], s_sh.at[dst_v[b]], ssems[b],
                             add=True)
        return carry

    lax.fori_loop(0, NCHUNK // NBUF, outer_body, 0)
    for b in range(NBUF):
        pltpu.make_async_copy(rows_v.at[b], s_sh.at[dst_v[b]],
                              ssems[b]).wait()
    plsc.subcore_barrier()

    def out_body(j, carry):
        cid = s + j * NS

        @pl.when(cid < N_ROW_CHUNKS)
        def _():
            r0 = cid * ROW_CHUNK
            pltpu.sync_copy(s_sh.at[pl.ds(r0, ROW_CHUNK)],
                            out_hbm.at[pl.ds(c * N + r0, ROW_CHUNK)])
        return carry

    lax.fori_loop(0, (N_ROW_CHUNKS + NS - 1) // NS, out_body, 0)


_GRID = 10
_BLK = N // _GRID

_row_spec = pl.BlockSpec((_BLK, D), lambda i: (i, 0))
_s0_spec = pl.BlockSpec((_BLK, D), lambda i: (i, 0))
_s1_spec = pl.BlockSpec((_BLK, D), lambda i: (_GRID + i, 0))
_vec_spec = pl.BlockSpec((1, D), lambda i: (0, 0))
_w_spec = pl.BlockSpec((D, D), lambda i: (0, 0))
_dinv_spec = pl.BlockSpec((_BLK, 1), lambda i: (i, 0))

_DOT = dict(preferred_element_type=jnp.float32,
            precision=lax.Precision.HIGHEST)


def _pre_body(x_ref, hist_ref, g_ref, be_ref, ew_ref, eb_ref, w0_ref,
              h1_ref, u0_ref, dinv_ref):
    deg = jnp.sum(hist_ref[...], axis=1, keepdims=True) + 1.0
    dinv = lax.rsqrt(deg)
    xb = x_ref[...] * (g_ref[...] * _BN_SCALE) + be_ref[...]
    h1 = jnp.dot(xb, ew_ref[...], **_DOT) + eb_ref[...]
    h1_ref[...] = h1
    u0_ref[...] = dinv * jnp.dot(h1, w0_ref[...], **_DOT)
    dinv_ref[...] = dinv


_tc_pre = pl.pallas_call(
    _pre_body,
    grid=(_GRID,),
    in_specs=[_row_spec,
              pl.BlockSpec((_BLK, NW), lambda i: (i, 0)),
              _vec_spec, _vec_spec, _w_spec, _vec_spec, _w_spec],
    out_specs=[_row_spec, _row_spec, _dinv_spec],
    out_shape=[jax.ShapeDtypeStruct((N, D), jnp.float32),
               jax.ShapeDtypeStruct((N, D), jnp.float32),
               jax.ShapeDtypeStruct((N, 1), jnp.float32)],
)


def _layer_update(h, u, s0, s1, dinv, b, lg, lb):
    z = dinv * (s0 + s1 + u) + b
    mu = jnp.mean(z, axis=1, keepdims=True)
    zc = z - mu
    var = jnp.mean(zc * zc, axis=1, keepdims=True)
    zn = zc * lax.rsqrt(var + 1e-5) * lg + lb
    zn = jnp.where(zn >= 0, zn, 0.1 * zn)
    return zn + h


def _mid_body(h_ref, s0_ref, s1_ref, dinv_ref, wl_ref, b_ref, lg_ref,
              lb_ref, wn_ref, hn_ref, un_ref):
    dinv = dinv_ref[...]
    h = h_ref[...]
    u = dinv * jnp.dot(h, wl_ref[...], **_DOT)
    hn = _layer_update(h, u, s0_ref[...], s1_ref[...],
                       dinv, b_ref[...], lg_ref[...], lb_ref[...])
    hn_ref[...] = hn
    un_ref[...] = dinv * jnp.dot(hn, wn_ref[...], **_DOT)


_tc_mid = pl.pallas_call(
    _mid_body,
    grid=(_GRID,),
    in_specs=[_row_spec, _s0_spec, _s1_spec, _dinv_spec, _w_spec,
              _vec_spec, _vec_spec, _vec_spec, _w_spec],
    out_specs=[_row_spec, _row_spec],
    out_shape=[jax.ShapeDtypeStruct((N, D), jnp.float32),
               jax.ShapeDtypeStruct((N, D), jnp.float32)],
)


def _post_body(h_ref, s0_ref, s1_ref, dinv_ref, wl_ref, b_ref, lg_ref,
               lb_ref, f1w_ref, f1b_ref, fg_ref, fb_ref, f2w_ref, f2b_ref,
               out_ref, acc_ref):
    i = pl.program_id(0)
    dinv = dinv_ref[...]
    h = h_ref[...]
    u = dinv * jnp.dot(h, wl_ref[...], **_DOT)
    hn = _layer_update(h, u, s0_ref[...], s1_ref[...],
                       dinv, b_ref[...], lg_ref[...], lb_ref[...])
    part = jnp.sum(hn, axis=0, keepdims=True)

    @pl.when(i == 0)
    def _():
        acc_ref[...] = part

    @pl.when(i > 0)
    def _():
        acc_ref[...] = acc_ref[...] + part

    @pl.when(i == _GRID - 1)
    def _():
        pooled = acc_ref[...] * (1.0 / N)
        y = jnp.dot(pooled, f1w_ref[...], **_DOT) + f1b_ref[...]
        mu = jnp.mean(y, axis=1, keepdims=True)
        yc = y - mu
        var = jnp.mean(yc * yc, axis=1, keepdims=True)
        yn = yc * lax.rsqrt(var + 1e-5) * fg_ref[...] + fb_ref[...]
        yn = jnp.where(yn >= 0, yn, 0.1 * yn)
        out_ref[...] = jnp.dot(yn, f2w_ref[...], **_DOT) + f2b_ref[...]


_tc_post = pl.pallas_call(
    _post_body,
    grid=(_GRID,),
    in_specs=[_row_spec, _s0_spec, _s1_spec, _dinv_spec, _w_spec,
              _vec_spec, _vec_spec, _vec_spec,
              _w_spec, _vec_spec, _vec_spec, _vec_spec, _w_spec, _vec_spec],
    out_specs=pl.BlockSpec((1, D), lambda i: (0, 0)),
    out_shape=jax.ShapeDtypeStruct((1, D), jnp.float32),
    scratch_shapes=[pltpu.VMEM((1, D), jnp.float32)],
)


def kernel(x, edge_index, bn_gamma, bn_beta, emb_W, emb_b, W0, b0, W1, b1,
           W2, b2, ln0_g, ln0_b, ln1_g, ln1_b, ln2_g, ln2_b, fc1_W, fc1_b,
           fcn_g, fcn_b, fc2_W, fc2_b):
    src = edge_index[0]
    dst = edge_index[1]
    pk = src + dst * 65536
    pads = jnp.broadcast_to(
        (N + jnp.arange(EPAD // NW - EPT, dtype=jnp.int32) % 8) * 65536,
        (NW, EPAD // NW - EPT))
    pk3 = jnp.concatenate([pk.reshape(NW, EPT), pads], axis=1)
    zeros_tile = jnp.zeros((ROW_CHUNK, D), jnp.float32)

    r = lambda v: v.reshape(1, D)

    hist = _sc_degree(dst).reshape(NW, N).T

    h1, u0, dinv = _tc_pre(x, hist, r(bn_gamma), r(bn_beta), emb_W,
                           r(emb_b), W0)

    s = _sc_aggregate(u0, pk3, zeros_tile)
    h2, u1 = _tc_mid(h1, s, s, dinv, W0, r(b0), r(ln0_g), r(ln0_b), W1)

    s = _sc_aggregate(u1, pk3, zeros_tile)
    h3, u2 = _tc_mid(h2, s, s, dinv, W1, r(b1), r(ln1_g), r(ln1_b), W2)

    s = _sc_aggregate(u2, pk3, zeros_tile)
    out = _tc_post(h3, s, s, dinv, W2, r(b2), r(ln2_g), r(ln2_b),
                   fc1_W, r(fc1_b), r(fcn_g), r(fcn_b), fc2_W, r(fc2_b))
    return out


# speedup vs baseline: 1.1050x; 1.1050x over previous
import functools

import jax
import jax.numpy as jnp
import numpy as np
from jax import lax
from jax.experimental import pallas as pl
from jax.experimental.pallas import tpu as pltpu
from jax.experimental.pallas import tpu_sc as plsc

N = 10000
E = 320000
D = 128

NC = 2
NS = 16
NW = NC * NS
EPT = E // NW
DEG_CHUNK = 2000
AGG_CHUNK = 80
ROW_CHUNK = 80
N_ROW_CHUNKS = N // ROW_CHUNK

_BN_SCALE = float(1.0 / np.sqrt(1.0 + 1e-5))

_MESH = plsc.VectorSubcoreMesh(core_axis_name="c", subcore_axis_name="s")


@functools.partial(
    pl.kernel,
    out_type=jax.ShapeDtypeStruct((NW * N,), jnp.float32),
    mesh=_MESH,
    scratch_types=[
        pltpu.VMEM((DEG_CHUNK,), jnp.int32),
        pltpu.VMEM((N,), jnp.float32),
    ],
    compiler_params=pltpu.CompilerParams(needs_layout_passes=False),
)
def _sc_degree(dst_hbm, out_hbm, dst_v, hist_v):
    g = lax.axis_index("c") * NS + lax.axis_index("s")

    def zero_body(i, carry):
        hist_v[pl.ds(i * 16, 16)] = jnp.zeros((16,), jnp.float32)
        return carry

    lax.fori_loop(0, N // 16, zero_body, 0)

    ones = jnp.ones((16,), jnp.float32)

    def chunk_body(ci, carry):
        pltpu.sync_copy(dst_hbm.at[pl.ds(g * EPT + ci * DEG_CHUNK, DEG_CHUNK)],
                        dst_v)

        def inner(j, c2):
            idx = dst_v[pl.ds(j * 16, 16)]
            plsc.addupdate_scatter(hist_v, [idx], ones)
            return c2

        lax.fori_loop(0, DEG_CHUNK // 16, inner, 0)
        return carry

    lax.fori_loop(0, EPT // DEG_CHUNK, chunk_body, 0)
    pltpu.sync_copy(hist_v, out_hbm.at[pl.ds(g * N, N)])


NBUF = 3
ECHUNK = 96
NCHUNK = 105
EPAD = NW * NCHUNK * ECHUNK
_CHUNK_BYTES = ECHUNK * D * 4


@functools.partial(
    pl.kernel,
    out_type=jax.ShapeDtypeStruct((NC * N, D), jnp.float32),
    mesh=_MESH,
    scratch_types=[
        pltpu.VMEM_SHARED((N + 8, D), jnp.float32),
        pltpu.VMEM((NCHUNK * ECHUNK,), jnp.int32),
        [pltpu.VMEM((ECHUNK,), jnp.int32) for _ in range(NBUF)],
        [pltpu.VMEM((ECHUNK,), jnp.int32) for _ in range(NBUF)],
        pltpu.VMEM((NBUF, ECHUNK, D), jnp.float32),
        [pltpu.SemaphoreType.DMA] * NBUF,
        [pltpu.SemaphoreType.DMA] * NBUF,
    ],
)
def _sc_aggregate(u_hbm, pk_hbm, out_hbm,
                  s_sh, pk_st, src_v, dst_v, rows_v, gsems, ssems):
    c = lax.axis_index("c")
    s = lax.axis_index("s")
    g = c * NS + s

    pltpu.async_copy(pk_hbm.at[g], pk_st, gsems[0])

    def zrow(i, carry):
        rows_v[0, i // 8, pl.ds((i % 8) * 16, 16)] = jnp.zeros(
            (16,), jnp.float32)
        return carry

    lax.fori_loop(0, ROW_CHUNK * 8, zrow, 0)
    zsrc = rows_v.at[0, pl.ds(0, ROW_CHUNK)]

    def zero_body(j, carry):
        cid = s + j * NS

        @pl.when(cid < N_ROW_CHUNKS)
        def _():
            pltpu.async_copy(zsrc, s_sh.at[pl.ds(cid * ROW_CHUNK,
                                                 ROW_CHUNK)], ssems[0])
        return carry

    nz = (N_ROW_CHUNKS + NS - 1) // NS
    lax.fori_loop(0, nz, zero_body, 0)

    def zero_drain(j, carry):
        cid = s + j * NS

        @pl.when(cid < N_ROW_CHUNKS)
        def _():
            pltpu.make_async_copy(zsrc, s_sh.at[pl.ds(cid * ROW_CHUNK,
                                                      ROW_CHUNK)],
                                  ssems[0]).wait()
        return carry

    lax.fori_loop(0, nz, zero_drain, 0)
    pltpu.make_async_copy(pk_hbm.at[g], pk_st, gsems[0]).wait()
    plsc.subcore_barrier()

    def unpack(ci, b):
        for k in range(ECHUNK // 16):
            v = pk_st[pl.ds(ci * ECHUNK + k * 16, 16)]
            src_v[b][pl.ds(k * 16, 16)] = lax.bitwise_and(v, 0xFFFF)
            dst_v[b][pl.ds(k * 16, 16)] = lax.shift_right_logical(v, 16)

    def outer_body(o, carry):
        for b in range(NBUF):
            ci = o * NBUF + b

            @pl.when(o > 0)
            def _():
                pltpu.make_async_copy(rows_v.at[b], s_sh.at[dst_v[b]],
                                      ssems[b]).wait()
            unpack(ci, b)
            pltpu.async_copy(u_hbm.at[src_v[b]], rows_v.at[b], gsems[b])
        for b in range(NBUF):
            pltpu.make_async_copy(u_hbm.at[src_v[b]], rows_v.at[b],
                                  gsems[b]).wait()
            pltpu.async_copy(rows_v.at[b], s_sh.at[dst_v[b]], ssems[b],
                             add=True)
        return carry

    lax.fori_loop(0, NCHUNK // NBUF, outer_body, 0)
    for b in range(NBUF):
        pltpu.make_async_copy(rows_v.at[b], s_sh.at[dst_v[b]],
                              ssems[b]).wait()
    plsc.subcore_barrier()

    def out_body(j, carry):
        cid = s + j * NS

        @pl.when(cid < N_ROW_CHUNKS)
        def _():
            r0 = cid * ROW_CHUNK
            pltpu.sync_copy(s_sh.at[pl.ds(r0, ROW_CHUNK)],
                            out_hbm.at[pl.ds(c * N + r0, ROW_CHUNK)])
        return carry

    lax.fori_loop(0, (N_ROW_CHUNKS + NS - 1) // NS, out_body, 0)


_GRID = 10
_BLK = N // _GRID

_row_spec = pl.BlockSpec((_BLK, D), lambda i: (i, 0))
_s0_spec = pl.BlockSpec((_BLK, D), lambda i: (i, 0))
_s1_spec = pl.BlockSpec((_BLK, D), lambda i: (_GRID + i, 0))
_vec_spec = pl.BlockSpec((1, D), lambda i: (0, 0))
_w_spec = pl.BlockSpec((D, D), lambda i: (0, 0))
_dinv_spec = pl.BlockSpec((_BLK, 1), lambda i: (i, 0))

_DOT = dict(preferred_element_type=jnp.float32,
            precision=lax.Precision.HIGHEST)


def _pre_body(x_ref, hist_ref, g_ref, be_ref, ew_ref, eb_ref, w0_ref,
              h1_ref, u0_ref, dinv_ref):
    deg = jnp.sum(hist_ref[...], axis=1, keepdims=True) + 1.0
    dinv = lax.rsqrt(deg)
    xb = x_ref[...] * (g_ref[...] * _BN_SCALE) + be_ref[...]
    h1 = jnp.dot(xb, ew_ref[...], **_DOT) + eb_ref[...]
    h1_ref[...] = h1
    u0_ref[...] = dinv * jnp.dot(h1, w0_ref[...], **_DOT)
    dinv_ref[...] = dinv


_tc_pre = pl.pallas_call(
    _pre_body,
    grid=(_GRID,),
    in_specs=[_row_spec,
              pl.BlockSpec((_BLK, NW), lambda i: (i, 0)),
              _vec_spec, _vec_spec, _w_spec, _vec_spec, _w_spec],
    out_specs=[_row_spec, _row_spec, _dinv_spec],
    out_shape=[jax.ShapeDtypeStruct((N, D), jnp.float32),
               jax.ShapeDtypeStruct((N, D), jnp.float32),
               jax.ShapeDtypeStruct((N, 1), jnp.float32)],
)


def _layer_update(h, u, s0, s1, dinv, b, lg, lb):
    z = dinv * (s0 + s1 + u) + b
    mu = jnp.mean(z, axis=1, keepdims=True)
    zc = z - mu
    var = jnp.mean(zc * zc, axis=1, keepdims=True)
    zn = zc * lax.rsqrt(var + 1e-5) * lg + lb
    zn = jnp.where(zn >= 0, zn, 0.1 * zn)
    return zn + h


def _mid_body(h_ref, u_ref, s0_ref, s1_ref, dinv_ref, b_ref, lg_ref, lb_ref,
              wn_ref, hn_ref, un_ref):
    dinv = dinv_ref[...]
    hn = _layer_update(h_ref[...], u_ref[...], s0_ref[...], s1_ref[...],
                       dinv, b_ref[...], lg_ref[...], lb_ref[...])
    hn_ref[...] = hn
    un_ref[...] = dinv * jnp.dot(hn, wn_ref[...], **_DOT)


_tc_mid = pl.pallas_call(
    _mid_body,
    grid=(_GRID,),
    in_specs=[_row_spec, _row_spec, _s0_spec, _s1_spec, _dinv_spec,
              _vec_spec, _vec_spec, _vec_spec, _w_spec],
    out_specs=[_row_spec, _row_spec],
    out_shape=[jax.ShapeDtypeStruct((N, D), jnp.float32),
               jax.ShapeDtypeStruct((N, D), jnp.float32)],
)


def _post_body(h_ref, u_ref, s0_ref, s1_ref, dinv_ref, b_ref, lg_ref, lb_ref,
               f1w_ref, f1b_ref, fg_ref, fb_ref, f2w_ref, f2b_ref,
               out_ref, acc_ref):
    i = pl.program_id(0)
    hn = _layer_update(h_ref[...], u_ref[...], s0_ref[...], s1_ref[...],
                       dinv_ref[...], b_ref[...], lg_ref[...], lb_ref[...])
    part = jnp.sum(hn, axis=0, keepdims=True)

    @pl.when(i == 0)
    def _():
        acc_ref[...] = part

    @pl.when(i > 0)
    def _():
        acc_ref[...] = acc_ref[...] + part

    @pl.when(i == _GRID - 1)
    def _():
        pooled = acc_ref[...] * (1.0 / N)
        y = jnp.dot(pooled, f1w_ref[...], **_DOT) + f1b_ref[...]
        mu = jnp.mean(y, axis=1, keepdims=True)
        yc = y - mu
        var = jnp.mean(yc * yc, axis=1, keepdims=True)
        yn = yc * lax.rsqrt(var + 1e-5) * fg_ref[...] + fb_ref[...]
        yn = jnp.where(yn >= 0, yn, 0.1 * yn)
        out_ref[...] = jnp.dot(yn, f2w_ref[...], **_DOT) + f2b_ref[...]


_tc_post = pl.pallas_call(
    _post_body,
    grid=(_GRID,),
    in_specs=[_row_spec, _row_spec, _s0_spec, _s1_spec, _dinv_spec,
              _vec_spec, _vec_spec, _vec_spec,
              _w_spec, _vec_spec, _vec_spec, _vec_spec, _w_spec, _vec_spec],
    out_specs=pl.BlockSpec((1, D), lambda i: (0, 0)),
    out_shape=jax.ShapeDtypeStruct((1, D), jnp.float32),
    scratch_shapes=[pltpu.VMEM((1, D), jnp.float32)],
)


def kernel(x, edge_index, bn_gamma, bn_beta, emb_W, emb_b, W0, b0, W1, b1,
           W2, b2, ln0_g, ln0_b, ln1_g, ln1_b, ln2_g, ln2_b, fc1_W, fc1_b,
           fcn_g, fcn_b, fc2_W, fc2_b):
    src = edge_index[0]
    dst = edge_index[1]
    pk = src + dst * 65536
    pads = jnp.broadcast_to(
        (N + jnp.arange(EPAD // NW - EPT, dtype=jnp.int32) % 8) * 65536,
        (NW, EPAD // NW - EPT))
    pk3 = jnp.concatenate([pk.reshape(NW, EPT), pads], axis=1)

    r = lambda v: v.reshape(1, D)

    hist = _sc_degree(dst).reshape(NW, N).T

    h1, u0, dinv = _tc_pre(x, hist, r(bn_gamma), r(bn_beta), emb_W,
                           r(emb_b), W0)

    s = _sc_aggregate(u0, pk3)
    h2, u1 = _tc_mid(h1, u0, s, s, dinv, r(b0), r(ln0_g), r(ln0_b), W1)

    s = _sc_aggregate(u1, pk3)
    h3, u2 = _tc_mid(h2, u1, s, s, dinv, r(b1), r(ln1_g), r(ln1_b), W2)

    s = _sc_aggregate(u2, pk3)
    out = _tc_post(h3, u2, s, s, dinv, r(b2), r(ln2_g), r(ln2_b),
                   fc1_W, r(fc1_b), r(fcn_g), r(fcn_b), fc2_W, r(fc2_b))
    return out
